# baseline (speedup 1.0000x reference)
import jax
import jax.numpy as jnp
from jax.experimental import pallas as pl
from jax.experimental.pallas import tpu as pltpu

K = 1024
D = 64


def _vq_body(x_ref, w_ref, zsq_ref, esq_ref, out_ref):
    x = x_ref[...].reshape(-1, D)
    w = w_ref[...]
    prod = jax.lax.dot_general(
        x, w, (((1,), (1,)), ((), ())),
        preferred_element_type=jnp.float32)
    z_sq = zsq_ref[...]
    e_sq = esq_ref[...]
    dist = z_sq - 2.0 * prod + e_sq[None, :]
    m = jnp.min(dist, axis=1, keepdims=True)
    iota = jax.lax.broadcasted_iota(jnp.int32, dist.shape, 1)
    idx = jnp.min(jnp.where(dist == m, iota, K), axis=1)
    out_ref[...] = idx.reshape(out_ref.shape)


def kernel(z_e, W):
    B, S, d = z_e.shape
    flat = z_e.reshape(-1, d)
    z_sq = jnp.sum(flat * flat, axis=1, keepdims=True)
    e_sq = jnp.sum(W * W, axis=1)
    BLOCK_B = 4
    BN = BLOCK_B * S
    grid = (B // BLOCK_B,)
    out = pl.pallas_call(
        _vq_body,
        grid=grid,
        in_specs=[
            pl.BlockSpec((BLOCK_B, S, d), lambda i: (i, 0, 0)),
            pl.BlockSpec((K, d), lambda i: (0, 0)),
            pl.BlockSpec((BN, 1), lambda i: (i, 0)),
            pl.BlockSpec((K,), lambda i: (0,)),
        ],
        out_specs=pl.BlockSpec((BLOCK_B, 1, S), lambda i: (i, 0, 0)),
        out_shape=jax.ShapeDtypeStruct((B, 1, S), jnp.int32),
        compiler_params=pltpu.CompilerParams(
            dimension_semantics=("parallel",)),
    )(z_e, W, z_sq, e_sq)
    return out.reshape(B, S)


# speedup vs baseline: 1.4178x; 1.4178x over previous
import jax
import jax.numpy as jnp
from jax.experimental import pallas as pl
from jax.experimental.pallas import tpu as pltpu

K = 1024
D = 64
SLAB = 64
NSLAB = K // SLAB


def _vq_body(x_ref, w2_ref, zsq_ref, esq_ref, out_ref):
    x = x_ref[...]
    w2 = w2_ref[...]
    z_sq = zsq_ref[...]
    BN = x.shape[0]

    def slab_dist(j):
        wj = w2[j * SLAB:(j + 1) * SLAB, :]
        pj = jax.lax.dot_general(wj, x, (((1,), (1,)), ((), ())),
                                 preferred_element_type=jnp.float32)
        ej = esq_ref[j * SLAB:(j + 1) * SLAB, :]
        return (z_sq - pj) + ej

    m = slab_dist(0)
    c = jnp.zeros((SLAB, BN), jnp.int32)
    for j in range(1, NSLAB):
        d_j = slab_dist(j)
        lt = d_j < m
        m = jnp.where(lt, d_j, m)
        c = jnp.where(lt, j, c)

    m_col = jnp.min(m, axis=0, keepdims=True)
    r = jax.lax.broadcasted_iota(jnp.int32, (SLAB, BN), 0)
    kcand = jnp.where(m == m_col, c * SLAB + r, K)
    out_ref[...] = jnp.min(kcand, axis=0, keepdims=True)


def kernel(z_e, W):
    B, S, d = z_e.shape
    N = B * S
    flat = z_e.reshape(N, d)
    z_sq = jnp.sum(flat * flat, axis=1, keepdims=True)
    e_sq = jnp.sum(W * W, axis=1)
    w2 = W + W
    BN = 4 * S
    grid = (N // BN,)
    out = pl.pallas_call(
        _vq_body,
        grid=grid,
        in_specs=[
            pl.BlockSpec((BN, d), lambda i: (i, 0)),
            pl.BlockSpec((K, d), lambda i: (0, 0)),
            pl.BlockSpec((1, BN), lambda i: (0, i)),
            pl.BlockSpec((K, 1), lambda i: (0, 0)),
        ],
        out_specs=pl.BlockSpec((1, BN), lambda i: (0, i)),
        out_shape=jax.ShapeDtypeStruct((1, N), jnp.int32),
        compiler_params=pltpu.CompilerParams(
            dimension_semantics=("parallel",)),
    )(flat, w2, z_sq.reshape(1, N), e_sq.reshape(K, 1))
    return out.reshape(B, S)


# speedup vs baseline: 2.0188x; 1.4239x over previous
import jax
import jax.numpy as jnp
from jax.experimental import pallas as pl
from jax.experimental.pallas import tpu as pltpu

K = 1024
D = 64
SS = 256
NSS = K // SS
RG = 8
NRG = SS // RG


def _vq_body(x_ref, w2_ref, zsq_ref, esq_ref, out_ref):
    x = x_ref[...]
    z_sq = zsq_ref[...]
    BN = x.shape[0]

    m = None
    c = None
    for ss in range(NSS):
        w2s = w2_ref[ss * SS:(ss + 1) * SS, :]
        p = jax.lax.dot_general(w2s, x, (((1,), (1,)), ((), ())),
                                preferred_element_type=jnp.float32)
        for g in range(NRG):
            pg = p[g * RG:(g + 1) * RG, :]
            eg = esq_ref[ss * SS + g * RG:ss * SS + (g + 1) * RG, :]
            dg = (z_sq - pg) + eg
            if m is None:
                m = dg
                c = jnp.zeros((RG, BN), jnp.int32)
            else:
                lt = dg < m
                m = jnp.where(lt, dg, m)
                c = jnp.where(lt, ss * NRG + g, c)

    r = jax.lax.broadcasted_iota(jnp.int32, (RG, BN), 0)
    kfull = c * RG + r
    m_min = jnp.min(m, axis=0, keepdims=True)
    kcand = jnp.where(m == m_min, kfull, K)
    out_ref[...] = jnp.min(kcand, axis=0, keepdims=True)


def kernel(z_e, W):
    B, S, d = z_e.shape
    N = B * S
    flat = z_e.reshape(N, d)
    z_sq = jnp.sum(flat * flat, axis=1, keepdims=True)
    e_sq = jnp.sum(W * W, axis=1)
    w2 = W + W
    BN = 4 * S
    grid = (N // BN,)
    out = pl.pallas_call(
        _vq_body,
        grid=grid,
        in_specs=[
            pl.BlockSpec((BN, d), lambda i: (i, 0)),
            pl.BlockSpec((K, d), lambda i: (0, 0)),
            pl.BlockSpec((1, BN), lambda i: (0, i)),
            pl.BlockSpec((K, 1), lambda i: (0, 0)),
        ],
        out_specs=pl.BlockSpec((1, BN), lambda i: (0, i)),
        out_shape=jax.ShapeDtypeStruct((1, N), jnp.int32),
        compiler_params=pltpu.CompilerParams(
            dimension_semantics=("parallel",)),
    )(flat, w2, z_sq.reshape(1, N), e_sq.reshape(K, 1))
    return out.reshape(B, S)


# speedup vs baseline: 2.0294x; 1.0052x over previous
import jax
import jax.numpy as jnp
from jax.experimental import pallas as pl
from jax.experimental.pallas import tpu as pltpu

K = 1024
D = 64
SS = 512
NSS = K // SS
RG = 8
NRG = SS // RG


def _vq_body(x_ref, w2_ref, zsq_ref, esq_ref, out_ref):
    x = x_ref[...]
    z_sq = zsq_ref[...]
    BN = x.shape[0]

    m = None
    c = None
    for ss in range(NSS):
        w2s = w2_ref[ss * SS:(ss + 1) * SS, :]
        p = jax.lax.dot_general(w2s, x, (((1,), (1,)), ((), ())),
                                preferred_element_type=jnp.float32)
        for g in range(NRG):
            pg = p[g * RG:(g + 1) * RG, :]
            eg = esq_ref[ss * SS + g * RG:ss * SS + (g + 1) * RG, :]
            dg = (z_sq - pg) + eg
            if m is None:
                m = dg
                c = jnp.zeros((RG, BN), jnp.int32)
            else:
                lt = dg < m
                m = jnp.where(lt, dg, m)
                c = jnp.where(lt, ss * NRG + g, c)

    r = jax.lax.broadcasted_iota(jnp.int32, (RG, BN), 0)
    kfull = c * RG + r
    m_min = jnp.min(m, axis=0, keepdims=True)
    kcand = jnp.where(m == m_min, kfull, K)
    out_ref[...] = jnp.min(kcand, axis=0, keepdims=True)


def kernel(z_e, W):
    B, S, d = z_e.shape
    N = B * S
    flat = z_e.reshape(N, d)
    z_sq = jnp.sum(flat * flat, axis=1, keepdims=True)
    e_sq = jnp.sum(W * W, axis=1)
    w2 = W + W
    BN = 4 * S
    grid = (N // BN,)
    out = pl.pallas_call(
        _vq_body,
        grid=grid,
        in_specs=[
            pl.BlockSpec((BN, d), lambda i: (i, 0)),
            pl.BlockSpec((K, d), lambda i: (0, 0)),
            pl.BlockSpec((1, BN), lambda i: (0, i)),
            pl.BlockSpec((K, 1), lambda i: (0, 0)),
        ],
        out_specs=pl.BlockSpec((1, BN), lambda i: (0, i)),
        out_shape=jax.ShapeDtypeStruct((1, N), jnp.int32),
        compiler_params=pltpu.CompilerParams(
            dimension_semantics=("parallel",)),
    )(flat, w2, z_sq.reshape(1, N), e_sq.reshape(K, 1))
    return out.reshape(B, S)
